# baseline (speedup 1.0000x reference)
import functools

import jax
import jax.numpy as jnp
from jax import lax
from jax.experimental import pallas as pl
from jax.experimental.pallas import tpu as pltpu

_T = 16
_S_CHUNK = 32
_LANE_TILE = 256
_UNROLL = 8


def _lif_body(x_ref, y_ref, mem_ref, res_ref, *, s_chunk, unroll):
    sc = pl.program_id(1)

    @pl.when(sc == 0)
    def _init():
        mem_ref[...] = jnp.zeros_like(mem_ref)
        res_ref[...] = jnp.zeros_like(res_ref)

    one = jnp.float32(1.0)
    zero = jnp.float32(0.0)

    def step(t, carry):
        m, r = carry
        m = m + x_ref[t]
        c = m >= one
        r = 0.5 * r + jnp.where(c, one, zero)
        y_ref[t] = r
        m = jnp.where(c, zero, m)
        return m, r

    m, r = lax.fori_loop(0, s_chunk, step, (mem_ref[...], res_ref[...]),
                         unroll=unroll)
    mem_ref[...] = m
    res_ref[...] = r


def kernel(x):
    steps, TB, D = x.shape
    B = TB // _T
    S = steps * _T

    xk = x.reshape(S, B, D)

    td = _LANE_TILE if D % _LANE_TILE == 0 else D
    n_d = D // td
    s_chunk = _S_CHUNK if S % _S_CHUNK == 0 else S
    n_s = S // s_chunk

    body = functools.partial(_lif_body, s_chunk=s_chunk, unroll=_UNROLL)

    y, mem, res = pl.pallas_call(
        body,
        out_shape=(
            jax.ShapeDtypeStruct((S, B, D), jnp.float32),
            jax.ShapeDtypeStruct((B, D), jnp.float32),
            jax.ShapeDtypeStruct((B, D), jnp.float32),
        ),
        grid=(n_d, n_s),
        in_specs=[pl.BlockSpec((s_chunk, B, td), lambda j, s: (s, 0, j))],
        out_specs=(
            pl.BlockSpec((s_chunk, B, td), lambda j, s: (s, 0, j)),
            pl.BlockSpec((B, td), lambda j, s: (0, j)),
            pl.BlockSpec((B, td), lambda j, s: (0, j)),
        ),
        compiler_params=pltpu.CompilerParams(
            dimension_semantics=("parallel", "arbitrary"),
            vmem_limit_bytes=64 * 1024 * 1024,
        ),
    )(xk)

    return y.reshape(steps, TB, D), mem, res


# speedup vs baseline: 1.1433x; 1.1433x over previous
import functools

import jax
import jax.numpy as jnp
from jax import lax
from jax.experimental import pallas as pl
from jax.experimental.pallas import tpu as pltpu

_T = 16
_S_CHUNK = 128
_LANE_TILE = 256
_UNROLL = 8


def _lif_body(x_ref, y_ref, mem_ref, res_ref, *, s_chunk, unroll):
    sc = pl.program_id(1)

    @pl.when(sc == 0)
    def _init():
        mem_ref[...] = jnp.zeros_like(mem_ref)
        res_ref[...] = jnp.zeros_like(res_ref)

    one = jnp.float32(1.0)
    zero = jnp.float32(0.0)

    def step(t, carry):
        m, r = carry
        m = m + x_ref[t]
        c = m >= one
        r = 0.5 * r + jnp.where(c, one, zero)
        y_ref[t] = r
        m = jnp.where(c, zero, m)
        return m, r

    m, r = lax.fori_loop(0, s_chunk, step, (mem_ref[...], res_ref[...]),
                         unroll=unroll)
    mem_ref[...] = m
    res_ref[...] = r


def kernel(x):
    steps, TB, D = x.shape
    B = TB // _T
    S = steps * _T

    xk = x.reshape(S, B, D)

    td = _LANE_TILE if D % _LANE_TILE == 0 else D
    n_d = D // td
    s_chunk = _S_CHUNK if S % _S_CHUNK == 0 else S
    n_s = S // s_chunk

    body = functools.partial(_lif_body, s_chunk=s_chunk, unroll=_UNROLL)

    y, mem, res = pl.pallas_call(
        body,
        out_shape=(
            jax.ShapeDtypeStruct((S, B, D), jnp.float32),
            jax.ShapeDtypeStruct((B, D), jnp.float32),
            jax.ShapeDtypeStruct((B, D), jnp.float32),
        ),
        grid=(n_d, n_s),
        in_specs=[pl.BlockSpec((s_chunk, B, td), lambda j, s: (s, 0, j))],
        out_specs=(
            pl.BlockSpec((s_chunk, B, td), lambda j, s: (s, 0, j)),
            pl.BlockSpec((B, td), lambda j, s: (0, j)),
            pl.BlockSpec((B, td), lambda j, s: (0, j)),
        ),
        compiler_params=pltpu.CompilerParams(
            dimension_semantics=("parallel", "arbitrary"),
            vmem_limit_bytes=64 * 1024 * 1024,
        ),
    )(xk)

    return y.reshape(steps, TB, D), mem, res
